# Initial kernel scaffold; baseline (speedup 1.0000x reference)
import jax
import jax.numpy as jnp
from jax.experimental import pallas as pl


def kernel(inputs, edge_index, W0, b0, W1, b1):
    raise NotImplementedError("write your pallas kernel here")


# speedup vs baseline: 9.1702x; 9.1702x over previous
import functools

import jax
import jax.numpy as jnp
from jax import lax
from jax.experimental import pallas as pl
from jax.experimental.pallas import tpu as pltpu
from jax.experimental.pallas import tpu_sc as plsc

NC = 2
NS = 16
NW = NC * NS
K = 128
DEGW = 16


def _ceil_to(x, m):
    return -(-x // m) * m


@functools.lru_cache(maxsize=None)
def _build(n, e, d):
    npad = _ceil_to(n + 1, 256)
    rpt = npad // NW
    nfull, rem = divmod(rpt, 128)
    epad = _ceil_to(e, NW * K)
    chunks = epad // (NW * K)
    epw = epad // NW
    eps = epad // NC
    rb = max(b for b in (1024, 512, 256) if npad % b == 0)
    grid = npad // rb
    dl = d // 16

    mesh = plsc.VectorSubcoreMesh(core_axis_name="c", subcore_axis_name="s")

    @functools.partial(
        pl.kernel,
        out_type=jax.ShapeDtypeStruct((NC, npad, DEGW), jnp.float32),
        mesh=mesh,
        scratch_types=[
            pltpu.VMEM((K,), jnp.int32),
            pltpu.VMEM((K, DEGW), jnp.float32),
            pltpu.VMEM((rpt, DEGW), jnp.float32),
            pltpu.VMEM_SHARED((npad, DEGW), jnp.float32),
        ],
    )
    def _degree(src_hbm, out_hbm, idx_v, ones_v, zero_v, acc):
        c = lax.axis_index("c")
        s = lax.axis_index("s")

        def fill_ones(i, carry):
            ones_v[i, :] = jnp.ones((DEGW,), jnp.float32)
            return carry

        lax.fori_loop(0, K, fill_ones, 0)

        def fill_zero(i, carry):
            zero_v[i, :] = jnp.zeros((DEGW,), jnp.float32)
            return carry

        lax.fori_loop(0, rpt, fill_zero, 0)
        r0 = s * rpt
        pltpu.sync_copy(zero_v, acc.at[pl.ds(r0, rpt), :])
        plsc.subcore_barrier()

        base0 = c * eps + s * epw

        def eb(j, carry):
            base = pl.multiple_of(base0 + j * K, 8)
            pltpu.sync_copy(src_hbm.at[pl.ds(base, K)], idx_v)
            pltpu.sync_copy(ones_v, acc.at[idx_v], add=True)
            return carry

        lax.fori_loop(0, chunks, eb, 0)
        plsc.subcore_barrier()
        pltpu.sync_copy(acc.at[pl.ds(r0, rpt), :], out_hbm.at[c, pl.ds(r0, rpt), :])

    @functools.partial(
        pl.kernel,
        out_type=jax.ShapeDtypeStruct((NC, npad, d), jnp.float32),
        mesh=mesh,
        scratch_types=[
            pltpu.VMEM((K,), jnp.int32),
            pltpu.VMEM((K,), jnp.int32),
            pltpu.VMEM((K, d), jnp.float32),
            pltpu.VMEM((128, d), jnp.float32),
            pltpu.VMEM_SHARED((npad, d), jnp.float32),
            pltpu.SemaphoreType.DMA,
        ],
    )
    def _propagate(h_hbm, src_hbm, dst_hbm, out_hbm, src_v, dst_v, rows_v,
                   zero_v, acc, sem):
        c = lax.axis_index("c")
        s = lax.axis_index("s")

        def fz(i, carry):
            zero_v[i // dl, pl.ds((i % dl) * 16, 16)] = jnp.zeros((16,), jnp.float32)
            return carry

        lax.fori_loop(0, 128 * dl, fz, 0)
        r0 = s * rpt
        for b in range(nfull):
            pltpu.sync_copy(zero_v, acc.at[pl.ds(r0 + b * 128, 128), :])
        if rem:
            pltpu.sync_copy(zero_v.at[pl.ds(0, rem), :],
                            acc.at[pl.ds(r0 + nfull * 128, rem), :])
        plsc.subcore_barrier()

        base0 = c * eps + s * epw

        def eb(j, carry):
            base = pl.multiple_of(base0 + j * K, 8)
            pltpu.sync_copy(src_hbm.at[pl.ds(base, K)], src_v)
            pltpu.sync_copy(dst_hbm.at[pl.ds(base, K)], dst_v)
            pltpu.async_copy(h_hbm.at[src_v], rows_v, sem).wait()
            pltpu.sync_copy(rows_v, acc.at[dst_v], add=True)
            return carry

        lax.fori_loop(0, chunks, eb, 0)
        plsc.subcore_barrier()
        for b in range(nfull):
            pltpu.sync_copy(acc.at[pl.ds(r0 + b * 128, 128), :],
                            out_hbm.at[c, pl.ds(r0 + b * 128, 128), :])
        if rem:
            pltpu.sync_copy(acc.at[pl.ds(r0 + nfull * 128, rem), :],
                            out_hbm.at[c, pl.ds(r0 + nfull * 128, rem), :])

    def _fc0_body(x_ref, degs_ref, w_ref, b_ref, h_ref, dis_ref):
        deg2 = degs_ref[0, :, :1] + degs_ref[1, :, :1]
        dis2 = deg2 ** -0.5
        h = jnp.dot(x_ref[...], w_ref[...], preferred_element_type=jnp.float32)
        h_ref[...] = (h + b_ref[...][None, :]) * dis2
        dis_ref[...] = dis2[:, 0]

    _fc0 = pl.pallas_call(
        _fc0_body,
        grid=(grid,),
        in_specs=[
            pl.BlockSpec((rb, d), lambda i: (i, 0)),
            pl.BlockSpec((NC, rb, DEGW), lambda i: (0, i, 0)),
            pl.BlockSpec((d, d), lambda i: (0, 0)),
            pl.BlockSpec((d,), lambda i: (0,)),
        ],
        out_specs=[
            pl.BlockSpec((rb, d), lambda i: (i, 0)),
            pl.BlockSpec((rb,), lambda i: (i,)),
        ],
        out_shape=[
            jax.ShapeDtypeStruct((npad, d), jnp.float32),
            jax.ShapeDtypeStruct((npad,), jnp.float32),
        ],
    )

    def _fc1_body(p_ref, dis_ref, w_ref, b_ref, h_ref):
        dis2 = dis_ref[...][:, None]
        h1 = jnp.maximum(2.0 * dis2 * (p_ref[0] + p_ref[1]), 0.0)
        h = jnp.dot(h1, w_ref[...], preferred_element_type=jnp.float32)
        h_ref[...] = (h + b_ref[...][None, :]) * dis2

    _fc1 = pl.pallas_call(
        _fc1_body,
        grid=(grid,),
        in_specs=[
            pl.BlockSpec((NC, rb, d), lambda i: (0, i, 0)),
            pl.BlockSpec((rb,), lambda i: (i,)),
            pl.BlockSpec((d, d), lambda i: (0, 0)),
            pl.BlockSpec((d,), lambda i: (0,)),
        ],
        out_specs=pl.BlockSpec((rb, d), lambda i: (i, 0)),
        out_shape=jax.ShapeDtypeStruct((npad, d), jnp.float32),
    )

    def _final_body(p_ref, dis_ref, o_ref):
        o_ref[...] = (p_ref[0] + p_ref[1]) * dis_ref[...][:, None]

    _final = pl.pallas_call(
        _final_body,
        grid=(grid,),
        in_specs=[
            pl.BlockSpec((NC, rb, d), lambda i: (0, i, 0)),
            pl.BlockSpec((rb,), lambda i: (i,)),
        ],
        out_specs=pl.BlockSpec((rb, d), lambda i: (i, 0)),
        out_shape=jax.ShapeDtypeStruct((npad, d), jnp.float32),
    )

    def run(xpad, src, dst, w0t, b0, w1t, b1):
        degs = _degree(src)
        h0, dis = _fc0(xpad, degs, w0t, b0)
        p0 = _propagate(h0, src, dst)
        h1 = _fc1(p0, dis, w1t, b1)
        p1 = _propagate(h1, src, dst)
        return _final(p1, dis)

    return run, npad, epad


def kernel(inputs, edge_index, W0, b0, W1, b1):
    n, d = inputs.shape
    e = edge_index.shape[1]
    run, npad, epad = _build(n, e, d)
    src = jnp.concatenate(
        [edge_index[0].astype(jnp.int32), jnp.full((epad - e,), n, jnp.int32)])
    dst = jnp.concatenate(
        [edge_index[1].astype(jnp.int32), jnp.full((epad - e,), n, jnp.int32)])
    xpad = jnp.pad(inputs.astype(jnp.float32), ((0, npad - n), (0, 0)))
    out = run(xpad, src, dst, W0.T, b0, W1.T, b1)
    return out[:n]
